# Initial kernel scaffold; baseline (speedup 1.0000x reference)
import functools

import jax
import jax.numpy as jnp
from jax import lax
from jax.experimental import pallas as pl
from jax.experimental.pallas import tpu as pltpu
from jax.experimental.pallas import tpu_sc as plsc

_VOCAB = 1000000
_D = 32
_B = 4096
_L = 200
_TOT = _B * _L
_NC = 2
_NS = 16
_NW = _NC * _NS
_PER_W = _TOT // _NW
_G = 128
_K = 10
_CHUNK = _K * _G
_NCHUNK = _PER_W // _CHUNK


def _gather_kernel(idx_hbm, table_hbm, out_hbm, idx_v, rows_v, sem):
    wid = lax.axis_index("s") * _NC + lax.axis_index("c")
    base = wid * _PER_W
    gbase = wid * (_PER_W // _G)

    @pl.loop(0, _NCHUNK)
    def chunk_loop(c):
        roff = base + c * _CHUNK
        goff = gbase + c * _K
        pltpu.sync_copy(idx_hbm.at[pl.ds(goff, _K)], idx_v)
        for j in range(_K):
            pltpu.async_copy(
                table_hbm.at[idx_v.at[j]],
                rows_v.at[pl.ds(j * _G, _G)],
                sem,
            )
        pltpu.make_async_copy(table_hbm.at[pl.ds(0, _CHUNK)], rows_v, sem).wait()
        pltpu.sync_copy(rows_v, out_hbm.at[pl.ds(roff, _CHUNK)])


@jax.jit
def _embed_lookup(indices, table):
    idx2d = indices.reshape(_TOT // _G, _G)
    mesh = plsc.VectorSubcoreMesh(core_axis_name="c", subcore_axis_name="s")
    out = pl.kernel(
        _gather_kernel,
        out_type=jax.ShapeDtypeStruct((_TOT, _D), jnp.float32),
        mesh=mesh,
        scratch_types=[
            pltpu.VMEM((_K, _G), jnp.int32),
            pltpu.VMEM((_CHUNK, _D), jnp.float32),
            pltpu.SemaphoreType.DMA,
        ],
    )(idx2d, table)
    return out.reshape(_B, _L, _D)


def kernel(indices, table):
    return _embed_lookup(indices, table)


# speedup vs baseline: 1.4578x; 1.4578x over previous
import functools

import jax
import jax.numpy as jnp
from jax import lax
from jax.experimental import pallas as pl
from jax.experimental.pallas import tpu as pltpu
from jax.experimental.pallas import tpu_sc as plsc

_VOCAB = 1000000
_D = 32
_B = 4096
_L = 200
_TOT = _B * _L
_NC = 2
_NS = 16
_NW = _NC * _NS
_PER_W = _TOT // _NW
_G = 128
_K = 8
_CHUNK = _K * _G
_NCHUNK = _PER_W // _CHUNK


def _gather_kernel(idx_hbm, table_hbm, out_hbm, idx_v, rows_v, sem):
    wid = lax.axis_index("s") * _NC + lax.axis_index("c")
    base = wid * _PER_W
    gbase = wid * (_PER_W // _G)

    @pl.loop(0, _NCHUNK)
    def chunk_loop(c):
        roff = base + c * _CHUNK
        goff = gbase + c * _K
        pltpu.sync_copy(idx_hbm.at[pl.ds(goff, _K)], idx_v)
        for j in range(_K):
            pltpu.async_copy(
                table_hbm.at[idx_v.at[j]],
                rows_v.at[pl.ds(j * _G, _G)],
                sem,
            )
        pltpu.make_async_copy(table_hbm.at[pl.ds(0, _CHUNK)], rows_v, sem).wait()
        pltpu.sync_copy(rows_v, out_hbm.at[pl.ds(roff, _CHUNK)])


@jax.jit
def _embed_lookup(indices, table):
    idx2d = indices.reshape(_TOT // _G, _G)
    mesh = plsc.VectorSubcoreMesh(core_axis_name="c", subcore_axis_name="s")
    out = pl.kernel(
        _gather_kernel,
        out_type=jax.ShapeDtypeStruct((_TOT, _D), jnp.float32),
        mesh=mesh,
        scratch_types=[
            pltpu.VMEM((_K, _G), jnp.int32),
            pltpu.VMEM((_CHUNK, _D), jnp.float32),
            pltpu.SemaphoreType.DMA,
        ],
        compiler_params=pltpu.CompilerParams(use_tc_tiling_on_sc=False),
    )(idx2d, table)
    return out.reshape(_B, _L, _D)


def kernel(indices, table):
    return _embed_lookup(indices, table)


# speedup vs baseline: 1.5001x; 1.0290x over previous
import jax
import jax.numpy as jnp
from jax import lax
from jax.experimental import pallas as pl
from jax.experimental.pallas import tpu as pltpu
from jax.experimental.pallas import tpu_sc as plsc

_VOCAB = 1000000
_D = 32
_B = 4096
_L = 200
_TOT = _B * _L
_NC = 2
_NS = 16
_NW = _NC * _NS
_PER_W = _TOT // _NW
_G = 128
_K = 8
_CHUNK = _K * _G
_NCHUNK = _PER_W // _CHUNK
_GROWS = _PER_W // _G


def _gather_kernel(idx_hbm, table_hbm, out_hbm,
                   idx_v, rows0, rows1, sem0, sem1):
    wid = lax.axis_index("s") * _NC + lax.axis_index("c")
    base = wid * _PER_W
    gbase = wid * _GROWS

    rows = (rows0, rows1)
    sems = (sem0, sem1)

    pltpu.sync_copy(idx_hbm.at[pl.ds(gbase, _GROWS)], idx_v)

    def fire(b, c):
        for j in range(_K):
            pltpu.async_copy(
                table_hbm.at[idx_v.at[c * _K + j]],
                rows[b].at[pl.ds(j * _G, _G)],
                sems[b],
            )

    def drain_store(b, c):
        pltpu.make_async_copy(
            table_hbm.at[pl.ds(0, _CHUNK)], rows[b], sems[b]).wait()
        pltpu.sync_copy(rows[b], out_hbm.at[pl.ds(base + c * _CHUNK, _CHUNK)])

    fire(0, 0)
    fire(1, 1)

    @pl.loop(0, (_NCHUNK - 3) // 2)
    def steady(i):
        c = 2 * i
        drain_store(0, c)
        fire(0, c + 2)
        drain_store(1, c + 1)
        fire(1, c + 3)

    c_tail = _NCHUNK - 3
    drain_store(0, c_tail)
    fire(0, c_tail + 2)
    drain_store(1, c_tail + 1)
    drain_store(0, c_tail + 2)


@jax.jit
def _embed_lookup(indices, table):
    idx2d = indices.reshape(_TOT // _G, _G)
    mesh = plsc.VectorSubcoreMesh(core_axis_name="c", subcore_axis_name="s")
    out = pl.kernel(
        _gather_kernel,
        out_type=jax.ShapeDtypeStruct((_TOT, _D), jnp.float32),
        mesh=mesh,
        scratch_types=[
            pltpu.VMEM((_GROWS, _G), jnp.int32),
            pltpu.VMEM((_CHUNK, _D), jnp.float32),
            pltpu.VMEM((_CHUNK, _D), jnp.float32),
            pltpu.SemaphoreType.DMA,
            pltpu.SemaphoreType.DMA,
        ],
        compiler_params=pltpu.CompilerParams(use_tc_tiling_on_sc=False),
    )(idx2d, table)
    return out.reshape(_B, _L, _D)


def kernel(indices, table):
    return _embed_lookup(indices, table)


# speedup vs baseline: 1.5041x; 1.0026x over previous
import jax
import jax.numpy as jnp
from jax import lax
from jax.experimental import pallas as pl
from jax.experimental.pallas import tpu as pltpu
from jax.experimental.pallas import tpu_sc as plsc

_VOCAB = 1000000
_D = 32
_B = 4096
_L = 200
_NC = 2
_NS = 16
_NW = _NC * _NS
_ROWS_W = _B // _NW
_CB = 4
_NCH = _ROWS_W // _CB
_SPLIT = 104


def _gather_kernel(idx_hbm, table_hbm, out_hbm,
                   idx_v, rows0, rows1, sem0, sem1):
    wid = lax.axis_index("s") * _NC + lax.axis_index("c")
    base = wid * _ROWS_W

    rows = (rows0, rows1)
    sems = (sem0, sem1)

    pltpu.sync_copy(idx_hbm.at[pl.ds(base, _ROWS_W)], idx_v)

    def fire(b, c):
        for k in range(_CB):
            r = c * _CB + k
            pltpu.async_copy(
                table_hbm.at[idx_v.at[r, pl.ds(0, _SPLIT)]],
                rows[b].at[k, pl.ds(0, _SPLIT)],
                sems[b],
            )
            pltpu.async_copy(
                table_hbm.at[idx_v.at[r, pl.ds(_SPLIT, _L - _SPLIT)]],
                rows[b].at[k, pl.ds(_SPLIT, _L - _SPLIT)],
                sems[b],
            )

    def drain_store(b, c):
        pltpu.make_async_copy(
            out_hbm.at[pl.ds(0, _CB)], rows[b], sems[b]).wait()
        pltpu.sync_copy(rows[b], out_hbm.at[pl.ds(base + c * _CB, _CB)])

    fire(0, 0)
    fire(1, 1)

    @pl.loop(0, (_NCH - 2) // 2)
    def steady(i):
        c = 2 * i
        drain_store(0, c)
        fire(0, c + 2)
        drain_store(1, c + 1)
        fire(1, c + 3)

    drain_store(0, _NCH - 2)
    drain_store(1, _NCH - 1)


@jax.jit
def _embed_lookup(indices, table):
    mesh = plsc.VectorSubcoreMesh(core_axis_name="c", subcore_axis_name="s")
    out = pl.kernel(
        _gather_kernel,
        out_type=jax.ShapeDtypeStruct((_B, _L, _D), jnp.float32),
        mesh=mesh,
        scratch_types=[
            pltpu.VMEM((_ROWS_W, _L), jnp.int32),
            pltpu.VMEM((_CB, _L, _D), jnp.float32),
            pltpu.VMEM((_CB, _L, _D), jnp.float32),
            pltpu.SemaphoreType.DMA,
            pltpu.SemaphoreType.DMA,
        ],
        compiler_params=pltpu.CompilerParams(use_tc_tiling_on_sc=False),
    )(indices, table)
    return out


def kernel(indices, table):
    return _embed_lookup(indices, table)
